# Initial kernel scaffold; baseline (speedup 1.0000x reference)
import functools

import jax
import jax.numpy as jnp
from jax import lax
from jax.experimental import pallas as pl
from jax.experimental.pallas import tpu as pltpu
from jax.experimental.pallas import tpu_sc as plsc


def _build_kernel(B, S, D, V):
  N = B * S
  NC, NS = 2, 16
  NW = NC * NS
  RPW = N // NW
  G = 128
  C = 512
  GPC = C // G
  NCH = RPW // C
  XROWS = RPW // G
  OSTEP = C % S
  PBL = (S - 1) + C
  assert N % NW == 0 and RPW % C == 0 and C % G == 0

  mesh = plsc.VectorSubcoreMesh(core_axis_name="c", subcore_axis_name="s")

  @functools.partial(
      pl.kernel,
      mesh=mesh,
      out_type=jax.ShapeDtypeStruct((N, D), jnp.float32),
      scratch_types=dict(
          idx_v=pltpu.VMEM((RPW // G, G), jnp.int32),
          pb_v=pltpu.VMEM((PBL, D), jnp.float32),
          rows0=pltpu.VMEM((C, D), jnp.float32),
          rows1=pltpu.VMEM((C, D), jnp.float32),
          obuf0=pltpu.VMEM((C, D), jnp.float32),
          obuf1=pltpu.VMEM((C, D), jnp.float32),
          gsem0=pltpu.SemaphoreType.DMA,
          gsem1=pltpu.SemaphoreType.DMA,
          osem0=pltpu.SemaphoreType.DMA,
          osem1=pltpu.SemaphoreType.DMA,
      ),
  )
  def k(x_hbm, tok_hbm, pos_hbm, out_hbm, *, idx_v, pb_v, rows0, rows1,
        obuf0, obuf1, gsem0, gsem1, osem0, osem1):
    del tok_hbm
    wid = lax.axis_index("s") * NC + lax.axis_index("c")
    ibase = wid * (RPW // G)
    obase = wid * RPW

    pltpu.sync_copy(x_hbm.at[pl.ds(ibase, RPW // G)], idx_v)
    for kk in range(PBL // S):
      pltpu.sync_copy(pos_hbm, pb_v.at[pl.ds(kk * S, S)])
    if PBL % S:
      pltpu.sync_copy(pos_hbm.at[pl.ds(0, PBL % S)],
                      pb_v.at[pl.ds((PBL // S) * S, PBL % S)])

    rows = (rows0, rows1)
    obuf = (obuf0, obuf1)
    gsem = (gsem0, gsem1)
    osem = (osem0, osem1)

    def fire_gathers(g, b):
      for jj in range(GPC):
        pltpu.async_copy(
            k.refs[1].at[idx_v.at[g * GPC + jj]],
            rows[b].at[pl.ds(jj * G, G)],
            gsem[b],
        )

    return

  return k


def kernel(x, token_table, pos_table):
  B, S = x.shape
  V, D = token_table.shape
  k = _build_kernel(B, S, D, V)
  x2d = x.astype(jnp.int32).reshape(-1, 128)
  out = k(x2d, token_table, pos_table)
  return out.reshape(B, S, D)


# speedup vs baseline: 3.2694x; 3.2694x over previous
import functools

import jax
import jax.numpy as jnp
from jax import lax
from jax.experimental import pallas as pl
from jax.experimental.pallas import tpu as pltpu
from jax.experimental.pallas import tpu_sc as plsc


def _build_kernel(B, S, D, V):
  N = B * S
  NC, NS = 2, 16
  NW = NC * NS
  RPW = N // NW
  G = 128
  C = 512
  GPC = C // G
  NCH = RPW // C
  XROWS = RPW // G
  OSTEP = C % S
  PBL = (S - 1) + C
  PBL += (-PBL) % 8
  assert N % NW == 0 and RPW % C == 0 and C % G == 0 and NCH % 2 == 0
  assert S % 8 == 0 and (PBL % S) % 8 == 0

  mesh = plsc.VectorSubcoreMesh(core_axis_name="c", subcore_axis_name="s")

  @functools.partial(
      pl.kernel,
      mesh=mesh,
      out_type=jax.ShapeDtypeStruct((N, D), jnp.float32),
      scratch_types=dict(
          idx_v=pltpu.VMEM((XROWS, G), jnp.int32),
          pb_v=pltpu.VMEM((PBL, D), jnp.float32),
          rows0=pltpu.VMEM((C, D), jnp.float32),
          rows1=pltpu.VMEM((C, D), jnp.float32),
          obuf0=pltpu.VMEM((C, D), jnp.float32),
          obuf1=pltpu.VMEM((C, D), jnp.float32),
          gsem0=pltpu.SemaphoreType.DMA,
          gsem1=pltpu.SemaphoreType.DMA,
          osem0=pltpu.SemaphoreType.DMA,
          osem1=pltpu.SemaphoreType.DMA,
      ),
      compiler_params=pltpu.CompilerParams(use_tc_tiling_on_sc=False),
  )
  def k(x_hbm, tok_hbm, pos_hbm, out_hbm, *, idx_v, pb_v, rows0, rows1,
        obuf0, obuf1, gsem0, gsem1, osem0, osem1):
    wid = lax.axis_index("s") * NC + lax.axis_index("c")
    ibase = wid * XROWS
    obase = wid * RPW

    pltpu.sync_copy(x_hbm.at[pl.ds(ibase, XROWS)], idx_v)
    for kk in range(PBL // S):
      pltpu.sync_copy(pos_hbm, pb_v.at[pl.ds(kk * S, S)])
    if PBL % S:
      pltpu.sync_copy(pos_hbm.at[pl.ds(0, PBL % S)],
                      pb_v.at[pl.ds((PBL // S) * S, PBL % S)])

    rows = (rows0, rows1)
    obuf = (obuf0, obuf1)
    gsem = (gsem0, gsem1)
    osem = (osem0, osem1)

    def fire_gathers(g, b):
      for jj in range(GPC):
        pltpu.async_copy(
            tok_hbm.at[idx_v.at[g * GPC + jj]],
            rows[b].at[pl.ds(jj * G, G)],
            gsem[b],
        )

    def wait_gathers(b):
      pltpu.make_async_copy(
          out_hbm.at[pl.ds(0, C)], rows[b], gsem[b]).wait()

    def wait_out(b):
      pltpu.make_async_copy(
          obuf[b], out_hbm.at[pl.ds(0, C)], osem[b]).wait()

    fire_gathers(0, 0)
    fire_gathers(1, 1)

    def pair_body(i, o):
      for b in range(2):
        g = 2 * i + b
        rb, ob = rows[b], obuf[b]

        @pl.when(g >= 2)
        def _():
          wait_out(b)

        wait_gathers(b)

        oo = o

        @plsc.parallel_loop(0, C, unroll=8)
        def _(r):
          for h in range(D // 16):
            ob[r, pl.ds(h * 16, 16)] = (
                rb[r, pl.ds(h * 16, 16)] + pb_v[oo + r, pl.ds(h * 16, 16)])

        pltpu.async_copy(ob, out_hbm.at[pl.ds(obase + g * C, C)], osem[b])

        @pl.when(g + 2 < NCH)
        def _():
          fire_gathers(g + 2, b)

        o = jnp.where(o + OSTEP >= S, o + OSTEP - S, o + OSTEP)
      return o

    lax.fori_loop(0, NCH // 2, pair_body, jnp.int32(0))

    wait_out(0)
    wait_out(1)

  return k


def kernel(x, token_table, pos_table):
  B, S = x.shape
  V, D = token_table.shape
  k = _build_kernel(B, S, D, V)
  x2d = x.astype(jnp.int32).reshape(-1, 128)
  out = k(x2d, token_table, pos_table)
  return out.reshape(B, S, D)
